# Initial kernel scaffold; baseline (speedup 1.0000x reference)
import functools

import jax
import jax.numpy as jnp
from jax import lax
from jax.experimental import pallas as pl
from jax.experimental.pallas import tpu as pltpu
from jax.experimental.pallas import tpu_sc as plsc

CHUNK = 1024


def kernel(segment_ids, table):
    batch, num_fields = segment_ids.shape
    num_rows, d_model = table.shape
    n = batch * num_fields
    ids_flat = segment_ids.reshape(n).astype(jnp.int32)

    info = plsc.get_sparse_core_info()
    num_workers = info.num_cores * info.num_subcores
    per_worker = n // num_workers
    num_chunks = per_worker // CHUNK

    mesh = plsc.VectorSubcoreMesh(core_axis_name="c", subcore_axis_name="s")

    @functools.partial(
        pl.kernel,
        out_type=jax.ShapeDtypeStruct((n, d_model), jnp.float32),
        mesh=mesh,
        scratch_types=[
            pltpu.VMEM((CHUNK,), jnp.int32),
            pltpu.VMEM((CHUNK, d_model), jnp.float32),
            pltpu.SemaphoreType.DMA,
        ],
    )
    def gather_kernel(ids_hbm, table_hbm, out_hbm, idx_v, rows_v, sem):
        wid = lax.axis_index("s") * info.num_cores + lax.axis_index("c")
        base = wid * per_worker

        def body(c, carry):
            off = base + c * CHUNK
            pltpu.sync_copy(ids_hbm.at[pl.ds(off, CHUNK)], idx_v)
            pltpu.async_copy(table_hbm.at[idx_v], rows_v, sem).wait()
            pltpu.sync_copy(rows_v, out_hbm.at[pl.ds(off, CHUNK)])
            return carry

        lax.fori_loop(0, num_chunks, body, 0)

    out = gather_kernel(ids_flat, table)
    return out.reshape(batch, num_fields, d_model)


# speedup vs baseline: 1.9649x; 1.9649x over previous
import functools

import jax
import jax.numpy as jnp
from jax import lax
from jax.experimental import pallas as pl
from jax.experimental.pallas import tpu as pltpu
from jax.experimental.pallas import tpu_sc as plsc

CHUNK = 1024


def kernel(segment_ids, table):
    batch, num_fields = segment_ids.shape
    num_rows, d_model = table.shape
    n = batch * num_fields
    ids_flat = segment_ids.reshape(n).astype(jnp.int32)

    info = plsc.get_sparse_core_info()
    num_workers = info.num_cores * info.num_subcores
    per_worker = n // num_workers
    num_chunks = per_worker // CHUNK

    mesh = plsc.VectorSubcoreMesh(core_axis_name="c", subcore_axis_name="s")

    @functools.partial(
        pl.kernel,
        out_type=jax.ShapeDtypeStruct((n, d_model), jnp.float32),
        mesh=mesh,
        scratch_types=[
            pltpu.VMEM((CHUNK,), jnp.int32),
            pltpu.VMEM((CHUNK, d_model), jnp.float32),
            pltpu.SemaphoreType.DMA,
        ],
        compiler_params=pltpu.CompilerParams(use_tc_tiling_on_sc=False),
    )
    def gather_kernel(ids_hbm, table_hbm, out_hbm, idx_v, rows_v, sem):
        wid = lax.axis_index("s") * info.num_cores + lax.axis_index("c")
        base = wid * per_worker

        def body(c, carry):
            off = base + c * CHUNK
            pltpu.sync_copy(ids_hbm.at[pl.ds(off, CHUNK)], idx_v)
            pltpu.async_copy(table_hbm.at[idx_v], rows_v, sem).wait()
            pltpu.sync_copy(rows_v, out_hbm.at[pl.ds(off, CHUNK)])
            return carry

        lax.fori_loop(0, num_chunks, body, 0)

    out = gather_kernel(ids_flat, table)
    return out.reshape(batch, num_fields, d_model)


# speedup vs baseline: 1.9840x; 1.0097x over previous
import functools

import jax
import jax.numpy as jnp
from jax import lax
from jax.experimental import pallas as pl
from jax.experimental.pallas import tpu as pltpu
from jax.experimental.pallas import tpu_sc as plsc

CHUNK = 512
NBUF = 4
GSKEW = 2


def kernel(segment_ids, table):
    batch, num_fields = segment_ids.shape
    num_rows, d_model = table.shape
    n = batch * num_fields
    ids_flat = segment_ids.reshape(n).astype(jnp.int32)

    info = plsc.get_sparse_core_info()
    num_workers = info.num_cores * info.num_subcores
    per_worker = n // num_workers
    num_chunks = per_worker // CHUNK
    num_groups = num_chunks // NBUF

    mesh = plsc.VectorSubcoreMesh(core_axis_name="c", subcore_axis_name="s")

    @functools.partial(
        pl.kernel,
        out_type=jax.ShapeDtypeStruct((n, d_model), jnp.float32),
        mesh=mesh,
        scratch_types=(
            [pltpu.VMEM((per_worker,), jnp.int32),
             pltpu.VMEM((NBUF, CHUNK, d_model), jnp.float32)]
            + [pltpu.SemaphoreType.DMA] * (2 * NBUF)
        ),
        compiler_params=pltpu.CompilerParams(use_tc_tiling_on_sc=False),
    )
    def gather_kernel(ids_hbm, table_hbm, out_hbm, idx_all, rows_v, *sems):
        sem_g = sems[:NBUF]
        sem_w = sems[NBUF:]
        wid = lax.axis_index("s") * info.num_cores + lax.axis_index("c")
        base = wid * per_worker

        def g_desc(c, b):
            return pltpu.make_async_copy(
                table_hbm.at[idx_all.at[pl.ds(c * CHUNK, CHUNK)]],
                rows_v.at[b],
                sem_g[b])

        def w_desc(c, b):
            return pltpu.make_async_copy(
                rows_v.at[b],
                out_hbm.at[pl.ds(base + c * CHUNK, CHUNK)],
                sem_w[b])

        pltpu.sync_copy(ids_hbm.at[pl.ds(base, per_worker)], idx_all)

        for c in range(NBUF):
            g_desc(c, c).start()
            if c >= GSKEW:
                g_desc(c - GSKEW, c - GSKEW).wait()
                w_desc(c - GSKEW, c - GSKEW).start()

        def body(g, carry):
            for b in range(NBUF):
                c = g * NBUF + b
                w_desc(c - NBUF, b).wait()
                g_desc(c, b).start()
                cb = (b - GSKEW) % NBUF
                g_desc(c - GSKEW, cb).wait()
                w_desc(c - GSKEW, cb).start()
            return carry

        lax.fori_loop(1, num_groups, body, 0)

        for t in range(num_chunks - GSKEW, num_chunks):
            g_desc(t, t % NBUF).wait()
            w_desc(t, t % NBUF).start()
        for t in range(num_chunks - NBUF, num_chunks):
            w_desc(t, t % NBUF).wait()

    out = gather_kernel(ids_flat, table)
    return out.reshape(batch, num_fields, d_model)


# speedup vs baseline: 10.0353x; 5.0580x over previous
import functools

import jax
import jax.numpy as jnp
from jax import lax
from jax.experimental import pallas as pl
from jax.experimental.pallas import tpu as pltpu
from jax.experimental.pallas import tpu_sc as plsc

NBUF = 8
GSKEW = 4


def kernel(segment_ids, table):
    batch, num_fields = segment_ids.shape
    num_rows, d_model = table.shape
    ids = segment_ids.astype(jnp.int32)

    info = plsc.get_sparse_core_info()
    num_workers = info.num_cores * info.num_subcores
    b_per_worker = batch // num_workers
    num_groups = b_per_worker // NBUF

    mesh = plsc.VectorSubcoreMesh(core_axis_name="c", subcore_axis_name="s")

    @functools.partial(
        pl.kernel,
        out_type=jax.ShapeDtypeStruct((batch, num_fields, d_model), jnp.float32),
        mesh=mesh,
        scratch_types=(
            [pltpu.VMEM((b_per_worker, num_fields), jnp.int32),
             pltpu.VMEM((NBUF, num_fields, d_model), jnp.float32)]
            + [pltpu.SemaphoreType.DMA] * (2 * NBUF)
        ),
        compiler_params=pltpu.CompilerParams(use_tc_tiling_on_sc=False),
    )
    def gather_kernel(ids_hbm, table_hbm, out_hbm, idx_all, rows_v, *sems):
        sem_g = sems[:NBUF]
        sem_w = sems[NBUF:]
        wid = lax.axis_index("s") * info.num_cores + lax.axis_index("c")
        base = wid * b_per_worker

        def g_desc(bb, s):
            return pltpu.make_async_copy(
                table_hbm.at[idx_all.at[bb]],
                rows_v.at[s],
                sem_g[s])

        def w_desc(bb, s):
            return pltpu.make_async_copy(
                rows_v.at[s],
                out_hbm.at[base + bb],
                sem_w[s])

        pltpu.sync_copy(ids_hbm.at[pl.ds(base, b_per_worker)], idx_all)

        for c in range(NBUF):
            g_desc(c, c).start()
            if c >= GSKEW:
                g_desc(c - GSKEW, c - GSKEW).wait()
                w_desc(c - GSKEW, c - GSKEW).start()

        def body(g, carry):
            for s in range(NBUF):
                c = g * NBUF + s
                w_desc(c - NBUF, s).wait()
                g_desc(c, s).start()
                cs = (s - GSKEW) % NBUF
                g_desc(c - GSKEW, cs).wait()
                w_desc(c - GSKEW, cs).start()
            return carry

        lax.fori_loop(1, num_groups, body, 0)

        total = b_per_worker
        for t in range(total - GSKEW, total):
            g_desc(t, t % NBUF).wait()
            w_desc(t, t % NBUF).start()
        for t in range(total - NBUF, total):
            w_desc(t, t % NBUF).wait()

    return gather_kernel(ids, table)


# speedup vs baseline: 22.1833x; 2.2105x over previous
import functools

import jax
import jax.numpy as jnp
from jax import lax
from jax.experimental import pallas as pl
from jax.experimental.pallas import tpu as pltpu
from jax.experimental.pallas import tpu_sc as plsc

CH = 2048
NB = 4


def kernel(segment_ids, table):
    batch, num_fields = segment_ids.shape
    num_rows, d_model = table.shape
    ids_t = segment_ids.astype(jnp.int32).T
    table_t = table.T

    info = plsc.get_sparse_core_info()
    num_workers = info.num_cores * info.num_subcores

    chunks_per_f = batch // CH
    total = num_fields * chunks_per_f
    num_groups = total // NB
    tile_rows = num_fields * (d_model // 8) * (batch // 128)

    mesh = plsc.VectorSubcoreMesh(core_axis_name="c", subcore_axis_name="s")

    @functools.partial(
        pl.kernel,
        out_type=jax.ShapeDtypeStruct((tile_rows, 8, 128), jnp.float32),
        mesh=mesh,
        scratch_types=(
            [pltpu.VMEM((num_rows,), jnp.float32),
             pltpu.VMEM((NB, CH), jnp.int32),
             pltpu.VMEM((NB, CH // 128, 1, 128), jnp.float32),
             pltpu.SemaphoreType.DMA]
            + [pltpu.SemaphoreType.DMA] * (2 * NB)
        ),
        compiler_params=pltpu.CompilerParams(
            use_tc_tiling_on_sc=False, needs_layout_passes=False),
    )
    def gather_kernel(ids_hbm, tab_hbm, out_hbm, trow, ids_v, vals_v,
                      sem_t, *sems):
        sem_i = sems[:NB]
        sem_o = sems[NB:]
        wid = lax.axis_index("s") * info.num_cores + lax.axis_index("c")
        t_d = wid // 8
        s_sub = wid % 8

        def i_desc(c, sl):
            f = c // chunks_per_f
            cc = c % chunks_per_f
            return pltpu.make_async_copy(
                ids_hbm.at[f, pl.ds(cc * CH, CH)], ids_v.at[sl], sem_i[sl])

        def o_desc(c, sl):
            f = c // chunks_per_f
            cc = c % chunks_per_f
            r0 = f * (d_model // 8) * (batch // 128) + t_d * (batch // 128) \
                + cc * (CH // 128)
            return pltpu.make_async_copy(
                vals_v.at[sl],
                out_hbm.at[pl.ds(r0, CH // 128), pl.ds(s_sub, 1)],
                sem_o[sl])

        def compute(sl):
            iv = ids_v.at[sl]
            vv = vals_v.at[sl]

            def cbody(jj, carry):
                for u in range(16):
                    idx = iv[pl.ds(jj * 256 + u * 16, 16)]
                    vals = plsc.load_gather(trow, [idx])
                    vv[jj * 2 + u // 8, 0, pl.ds((u % 8) * 16, 16)] = vals
                return carry

            lax.fori_loop(0, (CH // 128) // 2, cbody, 0)

        pltpu.make_async_copy(tab_hbm.at[wid], trow, sem_t).start()
        for c in range(NB):
            i_desc(c, c).start()
        pltpu.make_async_copy(tab_hbm.at[wid], trow, sem_t).wait()
        for c in range(NB):
            i_desc(c, c).wait()
            compute(c)
            o_desc(c, c).start()
            i_desc(c + NB, c).start()

        def body(g, carry):
            for sl in range(NB):
                c = g * NB + sl
                o_desc(c - NB, sl).wait()
                i_desc(c, sl).wait()
                compute(sl)
                o_desc(c, sl).start()
                i_desc(c + NB, sl).start()
            return carry

        lax.fori_loop(1, num_groups - 1, body, 0)

        for c in range(total - NB, total):
            sl = c % NB
            o_desc(c - NB, sl).wait()
            i_desc(c, sl).wait()
            compute(sl)
            o_desc(c, sl).start()
        for c in range(total - NB, total):
            o_desc(c, c % NB).wait()

    out_lin = gather_kernel(ids_t, table_t)
    x = out_lin.reshape(num_fields, d_model // 8, batch // 128, 8, 128)
    y = x.transpose(2, 4, 0, 1, 3)
    return y.reshape(batch, num_fields, d_model)


# speedup vs baseline: 35.9906x; 1.6224x over previous
import functools

import jax
import jax.numpy as jnp
from jax import lax
from jax.experimental import pallas as pl
from jax.experimental.pallas import tpu as pltpu
from jax.experimental.pallas import tpu_sc as plsc

CH = 2048
NB = 4


def kernel(segment_ids, table):
    batch, num_fields = segment_ids.shape
    num_rows, d_model = table.shape
    ids_t = segment_ids.astype(jnp.int32).T
    table_t = table.T

    info = plsc.get_sparse_core_info()
    num_workers = info.num_cores * info.num_subcores

    chunks_per_f = batch // CH
    total = num_fields * chunks_per_f
    num_groups = total // NB
    tile_rows = num_fields * (d_model // 8) * (batch // 128)

    mesh = plsc.VectorSubcoreMesh(core_axis_name="c", subcore_axis_name="s")

    @functools.partial(
        pl.kernel,
        out_type=jax.ShapeDtypeStruct((tile_rows, 8, 128), jnp.float32),
        mesh=mesh,
        scratch_types=(
            [pltpu.VMEM((num_rows,), jnp.float32),
             pltpu.VMEM((NB, CH), jnp.int32),
             pltpu.VMEM((NB, CH // 128, 1, 128), jnp.float32),
             pltpu.SemaphoreType.DMA]
            + [pltpu.SemaphoreType.DMA] * (2 * NB)
        ),
        compiler_params=pltpu.CompilerParams(
            use_tc_tiling_on_sc=False, needs_layout_passes=False),
    )
    def gather_kernel(ids_hbm, tab_hbm, out_hbm, trow, ids_v, vals_v,
                      sem_t, *sems):
        sem_i = sems[:NB]
        sem_o = sems[NB:]
        wid = lax.axis_index("s") * info.num_cores + lax.axis_index("c")
        t_d = wid // 8
        s_sub = wid % 8

        def i_desc(c, sl):
            f = c // chunks_per_f
            cc = c % chunks_per_f
            return pltpu.make_async_copy(
                ids_hbm.at[f, pl.ds(cc * CH, CH)], ids_v.at[sl], sem_i[sl])

        def o_desc(c, sl):
            f = c // chunks_per_f
            cc = c % chunks_per_f
            r0 = f * (d_model // 8) * (batch // 128) + t_d * (batch // 128) \
                + cc * (CH // 128)
            return pltpu.make_async_copy(
                vals_v.at[sl],
                out_hbm.at[pl.ds(r0, CH // 128), pl.ds(s_sub, 1)],
                sem_o[sl])

        def compute(sl):
            iv = ids_v.at[sl]
            vv = vals_v.at[sl]

            @plsc.parallel_loop(0, CH // 16, step=1, unroll=8)
            def cbody(j):
                idx = iv[pl.ds(j * 16, 16)]
                vals = plsc.load_gather(trow, [idx])
                vv[j // 8, 0, pl.ds((j % 8) * 16, 16)] = vals

        pltpu.make_async_copy(tab_hbm.at[wid], trow, sem_t).start()
        for c in range(NB):
            i_desc(c, c).start()
        pltpu.make_async_copy(tab_hbm.at[wid], trow, sem_t).wait()
        for c in range(NB):
            i_desc(c, c).wait()
            compute(c)
            o_desc(c, c).start()
            i_desc(c + NB, c).start()

        def body(g, carry):
            for sl in range(NB):
                c = g * NB + sl
                o_desc(c - NB, sl).wait()
                i_desc(c, sl).wait()
                compute(sl)
                o_desc(c, sl).start()
                i_desc(c + NB, sl).start()
            return carry

        lax.fori_loop(1, num_groups - 1, body, 0)

        for c in range(total - NB, total):
            sl = c % NB
            o_desc(c - NB, sl).wait()
            i_desc(c, sl).wait()
            compute(sl)
            o_desc(c, sl).start()
        for c in range(total - NB, total):
            o_desc(c, c % NB).wait()

    out_lin = gather_kernel(ids_t, table_t)
    x = out_lin.reshape(num_fields, d_model // 8, batch // 128, 8, 128)
    y = x.transpose(2, 4, 0, 1, 3)
    return y.reshape(batch, num_fields, d_model)
